# baseline (speedup 1.0000x reference)
import functools

import jax
import jax.numpy as jnp
from jax import lax
from jax.experimental import pallas as pl
from jax.experimental.pallas import tpu as pltpu
from jax.experimental.pallas import tpu_sc as plsc

TOP_K = 2
NUM_EXPERTS = 16
N_TOK = 8192
D_MODEL = 2048

TBLK = 1024
NBLK = N_TOK // TBLK

NC, NS, LANES = 2, 16, 16
NW = NC * NS
TOK_PER_W = N_TOK // NW
CH = 32
NCHUNK = TOK_PER_W // CH
STRIP = 2048
NSTRIP = N_TOK * TOP_K // STRIP


def _tc_route_body(logits_ref, clean_ref, nstd_ref,
                   gates_ref, load_ref, part_ref, hist_ref, start_ref,
                   rlo_ref, rhi_ref, elo_ref, ehi_ref, glo_ref, ghi_ref,
                   tril_ref):
    i = pl.program_id(0)

    @pl.when(i == 0)
    def _init():
        load_ref[...] = jnp.zeros((NUM_EXPERTS,), jnp.float32)
        part_ref[...] = jnp.zeros((NUM_EXPERTS,), jnp.int32)
        hist_ref[...] = jnp.zeros((1, NUM_EXPERTS), jnp.int32)
        rowi = lax.broadcasted_iota(jnp.int32, (TBLK, TBLK), 0)
        colj = lax.broadcasted_iota(jnp.int32, (TBLK, TBLK), 1)
        tril_ref[...] = (colj < rowi).astype(jnp.bfloat16)

    logits = logits_ref[...]
    iota_e = lax.broadcasted_iota(jnp.int32, (TBLK, NUM_EXPERTS), 1)
    neginf = jnp.float32(-jnp.inf)

    m1 = jnp.max(logits, axis=1, keepdims=True)
    i1 = jnp.min(jnp.where(logits == m1, iota_e, NUM_EXPERTS),
                 axis=1, keepdims=True)
    l2 = jnp.where(iota_e == i1, neginf, logits)
    m2 = jnp.max(l2, axis=1, keepdims=True)
    i2 = jnp.min(jnp.where(l2 == m2, iota_e, NUM_EXPERTS),
                 axis=1, keepdims=True)
    l3 = jnp.where(iota_e == i2, neginf, l2)
    m3 = jnp.max(l3, axis=1, keepdims=True)

    e2v = jnp.exp(m2 - m1)
    g1 = 1.0 / (1.0 + e2v)
    g2 = e2v / (1.0 + e2v)

    clean = clean_ref[...]
    nstd = nstd_ref[...]
    inv_sqrt2 = jnp.float32(0.7071067811865476)
    th = jnp.where(logits > m3, m3, m2)
    prob = 0.5 * (1.0 + lax.erf((clean - th) / nstd * inv_sqrt2))
    load_ref[...] += prob.sum(axis=0)

    e_lo = jnp.minimum(i1, i2)
    e_hi = jnp.maximum(i1, i2)
    swap = i1 < i2
    g_lo = jnp.where(swap, g1, g2)
    g_hi = jnp.where(swap, g2, g1)
    ohlo = (iota_e == e_lo).astype(jnp.float32)
    ohhi = (iota_e == e_hi).astype(jnp.float32)
    pair_oh = ohlo + ohhi
    gates_ref[...] = g_lo * ohlo + g_hi * ohhi
    part_blk = ((g_lo > 0).astype(jnp.float32) * ohlo
                + (g_hi > 0).astype(jnp.float32) * ohhi).sum(axis=0)
    part_ref[...] += part_blk.astype(jnp.int32)

    excl = jax.lax.dot(tril_ref[...], pair_oh.astype(jnp.bfloat16),
                       preferred_element_type=jnp.float32)

    carry = hist_ref[...].astype(jnp.float32)
    ec = excl + carry
    rank_lo = (ohlo * ec).sum(axis=1)
    rank_hi = (ohhi * ec).sum(axis=1)
    hist_ref[...] += pair_oh.sum(axis=0).astype(jnp.int32)[None, :]

    rlo_ref[...] = rank_lo.astype(jnp.int32)
    rhi_ref[...] = rank_hi.astype(jnp.int32)
    elo_ref[...] = e_lo[:, 0]
    ehi_ref[...] = e_hi[:, 0]
    glo_ref[...] = g_lo[:, 0]
    ghi_ref[...] = g_hi[:, 0]

    @pl.when(i == NBLK - 1)
    def _start():
        h = hist_ref[...]
        zc = jnp.zeros((1, NUM_EXPERTS), jnp.int32)
        c = h
        for s in (1, 2, 4, 8):
            shifted = jnp.concatenate([zc[:, :s], c[:, :-s]], axis=1)
            c = c + shifted
        start_ref[...] = (c - h)[0, :]


def _tc_route(logits, clean_logits, noise_std):
    out_shape = (
        jax.ShapeDtypeStruct((N_TOK, NUM_EXPERTS), jnp.float32),
        jax.ShapeDtypeStruct((NUM_EXPERTS,), jnp.float32),
        jax.ShapeDtypeStruct((NUM_EXPERTS,), jnp.int32),
        jax.ShapeDtypeStruct((1, NUM_EXPERTS), jnp.int32),
        jax.ShapeDtypeStruct((NUM_EXPERTS,), jnp.int32),
        jax.ShapeDtypeStruct((N_TOK,), jnp.int32),
        jax.ShapeDtypeStruct((N_TOK,), jnp.int32),
        jax.ShapeDtypeStruct((N_TOK,), jnp.int32),
        jax.ShapeDtypeStruct((N_TOK,), jnp.int32),
        jax.ShapeDtypeStruct((N_TOK,), jnp.float32),
        jax.ShapeDtypeStruct((N_TOK,), jnp.float32),
    )
    blk_tok = pl.BlockSpec((TBLK, NUM_EXPERTS), lambda i: (i, 0))
    blk_one = pl.BlockSpec((1, NUM_EXPERTS), lambda i: (0, 0))
    blk_sm = pl.BlockSpec((NUM_EXPERTS,), lambda i: (0,))
    blk_vec = pl.BlockSpec((TBLK,), lambda i: (i,))
    return pl.pallas_call(
        _tc_route_body,
        grid=(NBLK,),
        in_specs=[blk_tok, blk_tok, blk_tok],
        out_specs=(blk_tok, blk_sm, blk_sm, blk_one, blk_sm,
                   blk_vec, blk_vec, blk_vec, blk_vec, blk_vec, blk_vec),
        out_shape=out_shape,
        scratch_shapes=[pltpu.VMEM((TBLK, TBLK), jnp.bfloat16)],
    )(logits, clean_logits, noise_std)


def _sc_dispatch_body(net_hbm, start_hbm, rlo_hbm, rhi_hbm, elo_hbm, ehi_hbm,
                      glo_hbm, ghi_hbm,
                      ei_hbm, bi_hbm, gg_hbm,
                      start_v, rlo_v, rhi_v, elo_v, ehi_v,
                      posrow_v, rows_v, strip_r, strip_e, strip_g,
                      dest_bi_v, dest_gg_v,
                      semr0, semr1, sema0, sema1, semb0, semb1):
    wid = lax.axis_index("s") * NC + lax.axis_index("c")
    base = wid * TOK_PER_W
    iota16 = lax.iota(jnp.int32, LANES)

    pltpu.sync_copy(start_hbm, start_v)

    pltpu.sync_copy(rlo_hbm.at[pl.ds(base, TOK_PER_W)], rlo_v)
    pltpu.sync_copy(rhi_hbm.at[pl.ds(base, TOK_PER_W)], rhi_v)
    pltpu.sync_copy(elo_hbm.at[pl.ds(base, TOK_PER_W)], elo_v)
    pltpu.sync_copy(ehi_hbm.at[pl.ds(base, TOK_PER_W)], ehi_v)

    for j in range(NCHUNK):
        for h in range(CH // LANES):
            off = j * CH + h * LANES
            rk = rlo_v[pl.ds(off, LANES)]
            ee = elo_v[pl.ds(off, LANES)]
            posrow_v[2 * j, pl.ds(h * LANES, LANES)] = \
                plsc.load_gather(start_v, [ee]) + rk
            rk = rhi_v[pl.ds(off, LANES)]
            ee = ehi_v[pl.ds(off, LANES)]
            posrow_v[2 * j + 1, pl.ds(h * LANES, LANES)] = \
                plsc.load_gather(start_v, [ee]) + rk

    def do_strip(s, is_bi):
        lo_half = s < NSTRIP // 2
        src_r = rlo_hbm if lo_half else rhi_hbm
        src_e = elo_hbm if lo_half else ehi_hbm
        src_g = glo_hbm if lo_half else ghi_hbm
        tok0 = (s % (NSTRIP // 2)) * STRIP
        pltpu.sync_copy(src_r.at[pl.ds(tok0, STRIP)], strip_r)
        pltpu.sync_copy(src_e.at[pl.ds(tok0, STRIP)], strip_e)
        if not is_bi:
            pltpu.sync_copy(src_g.at[pl.ds(tok0, STRIP)], strip_g)

        def body(k, carry):
            off = pl.multiple_of(k * LANES, 8)
            pos = plsc.load_gather(start_v, [strip_e[pl.ds(off, LANES)]]) \
                + strip_r[pl.ds(off, LANES)]
            if is_bi:
                plsc.store_scatter(dest_bi_v, [pos],
                                   iota16 + (tok0 + k * LANES))
            else:
                plsc.store_scatter(dest_gg_v, [pos],
                                   strip_g[pl.ds(off, LANES)])
            return carry

        lax.fori_loop(0, STRIP // LANES, body, 0)

    for j in range(NCHUNK):
        pltpu.sync_copy(net_hbm.at[pl.ds(base + j * CH, CH)], rows_v)
        c0 = pltpu.async_copy(rows_v, ei_hbm.at[posrow_v.at[2 * j]], sema0)
        c1 = pltpu.async_copy(rows_v, ei_hbm.at[posrow_v.at[2 * j + 1]],
                              semb0)
        if j < NSTRIP:
            strip_idx = j

            @pl.when(wid == 0)
            def _bi_strip():
                do_strip(strip_idx, True)

            @pl.when(wid == 1)
            def _gg_strip():
                do_strip(strip_idx, False)
        c0.wait()
        c1.wait()

    @pl.when(wid == 0)
    def _bi_out():
        pltpu.sync_copy(dest_bi_v, bi_hbm)

    @pl.when(wid == 1)
    def _gg_out():
        pltpu.sync_copy(dest_gg_v, gg_hbm)


def _sc_dispatch(net, start, rlo, rhi, elo, ehi, glo, ghi):
    mesh = plsc.VectorSubcoreMesh(core_axis_name="c", subcore_axis_name="s")
    fn = functools.partial(
        pl.kernel,
        out_type=(
            jax.ShapeDtypeStruct((N_TOK * TOP_K, D_MODEL), jnp.float32),
            jax.ShapeDtypeStruct((N_TOK * TOP_K,), jnp.int32),
            jax.ShapeDtypeStruct((N_TOK * TOP_K,), jnp.float32),
        ),
        mesh=mesh,
        compiler_params=pltpu.CompilerParams(needs_layout_passes=False),
        scratch_types=[
            pltpu.VMEM((NUM_EXPERTS,), jnp.int32),
            pltpu.VMEM((TOK_PER_W,), jnp.int32),
            pltpu.VMEM((TOK_PER_W,), jnp.int32),
            pltpu.VMEM((TOK_PER_W,), jnp.int32),
            pltpu.VMEM((TOK_PER_W,), jnp.int32),
            pltpu.VMEM((2 * NCHUNK, CH), jnp.int32),
            pltpu.VMEM((CH, D_MODEL), jnp.float32),
            pltpu.VMEM((STRIP,), jnp.int32),
            pltpu.VMEM((STRIP,), jnp.int32),
            pltpu.VMEM((STRIP,), jnp.float32),
            pltpu.VMEM((N_TOK * TOP_K,), jnp.int32),
            pltpu.VMEM((N_TOK * TOP_K,), jnp.float32),
            pltpu.SemaphoreType.DMA,
            pltpu.SemaphoreType.DMA,
            pltpu.SemaphoreType.DMA,
            pltpu.SemaphoreType.DMA,
            pltpu.SemaphoreType.DMA,
            pltpu.SemaphoreType.DMA,
        ],
    )(_sc_dispatch_body)
    return fn(net, start, rlo, rhi, elo, ehi, glo, ghi)


def kernel(net, logits, clean_logits, noise_std):
    (gates, load, part, hist, start,
     rlo, rhi, elo, ehi, glo, ghi) = _tc_route(logits, clean_logits,
                                               noise_std)
    del hist
    ei, bi, gg = _sc_dispatch(net, start, rlo, rhi, elo, ehi, glo, ghi)
    return gates, ei, bi, gg[:, None], load, part


# speedup vs baseline: 1.3455x; 1.3455x over previous
import functools

import jax
import jax.numpy as jnp
from jax import lax
from jax.experimental import pallas as pl
from jax.experimental.pallas import tpu as pltpu
from jax.experimental.pallas import tpu_sc as plsc

TOP_K = 2
NUM_EXPERTS = 16
N_TOK = 8192
D_MODEL = 2048

TBLK = 1024
NBLK = N_TOK // TBLK

NC, NS, LANES = 2, 16, 16
NW = NC * NS
TOK_PER_W = N_TOK // NW
CH = 16
NCHUNK = TOK_PER_W // CH
STRIP = 2048
NSTRIP = N_TOK * TOP_K // STRIP


def _tc_route_body(logits_ref, clean_ref, nstd_ref,
                   gates_ref, load_ref, part_ref, hist_ref, start_ref,
                   rlo_ref, rhi_ref, elo_ref, ehi_ref, glo_ref, ghi_ref,
                   triu_ref):
    i = pl.program_id(0)

    @pl.when(i == 0)
    def _init():
        load_ref[...] = jnp.zeros((NUM_EXPERTS,), jnp.float32)
        part_ref[...] = jnp.zeros((NUM_EXPERTS,), jnp.int32)
        hist_ref[...] = jnp.zeros((NUM_EXPERTS, 1), jnp.int32)
        rowi = lax.broadcasted_iota(jnp.int32, (TBLK, TBLK), 0)
        colj = lax.broadcasted_iota(jnp.int32, (TBLK, TBLK), 1)
        triu_ref[...] = (rowi < colj).astype(jnp.bfloat16)

    lgT = logits_ref[...].T
    iota_e = lax.broadcasted_iota(jnp.int32, (NUM_EXPERTS, TBLK), 0)
    neginf = jnp.float32(-jnp.inf)

    m1 = jnp.max(lgT, axis=0, keepdims=True)
    i1 = jnp.min(jnp.where(lgT == m1, iota_e, NUM_EXPERTS),
                 axis=0, keepdims=True)
    l2 = jnp.where(iota_e == i1, neginf, lgT)
    m2 = jnp.max(l2, axis=0, keepdims=True)
    i2 = jnp.min(jnp.where(l2 == m2, iota_e, NUM_EXPERTS),
                 axis=0, keepdims=True)
    l3 = jnp.where(iota_e == i2, neginf, l2)
    m3 = jnp.max(l3, axis=0, keepdims=True)

    e2v = jnp.exp(m2 - m1)
    g1 = 1.0 / (1.0 + e2v)
    g2 = e2v / (1.0 + e2v)

    cleanT = clean_ref[...].T
    nstdT = nstd_ref[...].T
    inv_sqrt2 = jnp.float32(0.7071067811865476)
    th = jnp.where(lgT > m3, m3, m2)
    prob = 0.5 * (1.0 + lax.erf((cleanT - th) / nstdT * inv_sqrt2))
    load_ref[...] += prob.sum(axis=1)

    e_lo = jnp.minimum(i1, i2)
    e_hi = jnp.maximum(i1, i2)
    swap = i1 < i2
    g_lo = jnp.where(swap, g1, g2)
    g_hi = jnp.where(swap, g2, g1)
    ohlo = (iota_e == e_lo).astype(jnp.float32)
    ohhi = (iota_e == e_hi).astype(jnp.float32)
    pair_oh = ohlo + ohhi
    gates_ref[...] = (g_lo * ohlo + g_hi * ohhi).T
    part_blk = ((g_lo > 0).astype(jnp.float32) * ohlo
                + (g_hi > 0).astype(jnp.float32) * ohhi).sum(axis=1)
    part_ref[...] += part_blk.astype(jnp.int32)

    excl = jax.lax.dot(pair_oh.astype(jnp.bfloat16), triu_ref[...],
                       preferred_element_type=jnp.float32)

    carry = hist_ref[...].astype(jnp.float32)
    ec = excl + carry
    rank_lo = (ohlo * ec).sum(axis=0)
    rank_hi = (ohhi * ec).sum(axis=0)
    hist_ref[...] += pair_oh.sum(axis=1, keepdims=True).astype(jnp.int32)

    rlo_ref[...] = rank_lo.astype(jnp.int32)
    rhi_ref[...] = rank_hi.astype(jnp.int32)
    elo_ref[...] = e_lo[0, :]
    ehi_ref[...] = e_hi[0, :]
    glo_ref[...] = g_lo[0, :]
    ghi_ref[...] = g_hi[0, :]

    @pl.when(i == NBLK - 1)
    def _start():
        h = hist_ref[...]
        zc = jnp.zeros((NUM_EXPERTS, 1), jnp.int32)
        c = h
        for s in (1, 2, 4, 8):
            shifted = jnp.concatenate([zc[:s, :], c[:-s, :]], axis=0)
            c = c + shifted
        start_ref[...] = (c - h)[:, 0]


def _tc_route(logits, clean_logits, noise_std):
    out_shape = (
        jax.ShapeDtypeStruct((N_TOK, NUM_EXPERTS), jnp.float32),
        jax.ShapeDtypeStruct((NUM_EXPERTS,), jnp.float32),
        jax.ShapeDtypeStruct((NUM_EXPERTS,), jnp.int32),
        jax.ShapeDtypeStruct((NUM_EXPERTS, 1), jnp.int32),
        jax.ShapeDtypeStruct((NUM_EXPERTS,), jnp.int32),
        jax.ShapeDtypeStruct((N_TOK,), jnp.int32),
        jax.ShapeDtypeStruct((N_TOK,), jnp.int32),
        jax.ShapeDtypeStruct((N_TOK,), jnp.int32),
        jax.ShapeDtypeStruct((N_TOK,), jnp.int32),
        jax.ShapeDtypeStruct((N_TOK,), jnp.float32),
        jax.ShapeDtypeStruct((N_TOK,), jnp.float32),
    )
    blk_tok = pl.BlockSpec((TBLK, NUM_EXPERTS), lambda i: (i, 0))
    blk_one = pl.BlockSpec((NUM_EXPERTS, 1), lambda i: (0, 0))
    blk_sm = pl.BlockSpec((NUM_EXPERTS,), lambda i: (0,))
    blk_vec = pl.BlockSpec((TBLK,), lambda i: (i,))
    return pl.pallas_call(
        _tc_route_body,
        grid=(NBLK,),
        in_specs=[blk_tok, blk_tok, blk_tok],
        out_specs=(blk_tok, blk_sm, blk_sm, blk_one, blk_sm,
                   blk_vec, blk_vec, blk_vec, blk_vec, blk_vec, blk_vec),
        out_shape=out_shape,
        scratch_shapes=[pltpu.VMEM((TBLK, TBLK), jnp.bfloat16)],
    )(logits, clean_logits, noise_std)


def _sc_dispatch_body(net_hbm, start_hbm, rlo_hbm, rhi_hbm, elo_hbm, ehi_hbm,
                      glo_hbm, ghi_hbm,
                      ei_hbm, bi_hbm, gg_hbm,
                      start_v, rlo_v, rhi_v, elo_v, ehi_v,
                      posrow_v, rows_v, strip_r, strip_e, strip_g,
                      dest_bi_v, dest_gg_v,
                      semr0, semr1, sema0, sema1, semb0, semb1):
    wid = lax.axis_index("s") * NC + lax.axis_index("c")
    base = wid * TOK_PER_W
    iota16 = lax.iota(jnp.int32, LANES)

    pltpu.sync_copy(start_hbm, start_v)

    pltpu.sync_copy(rlo_hbm.at[pl.ds(base, TOK_PER_W)], rlo_v)
    pltpu.sync_copy(rhi_hbm.at[pl.ds(base, TOK_PER_W)], rhi_v)
    pltpu.sync_copy(elo_hbm.at[pl.ds(base, TOK_PER_W)], elo_v)
    pltpu.sync_copy(ehi_hbm.at[pl.ds(base, TOK_PER_W)], ehi_v)

    for j in range(NCHUNK):
        off = j * CH
        rk = rlo_v[pl.ds(off, LANES)]
        ee = elo_v[pl.ds(off, LANES)]
        posrow_v[2 * j, :] = plsc.load_gather(start_v, [ee]) + rk
        rk = rhi_v[pl.ds(off, LANES)]
        ee = ehi_v[pl.ds(off, LANES)]
        posrow_v[2 * j + 1, :] = plsc.load_gather(start_v, [ee]) + rk

    def do_strip(s, is_bi):
        lo_half = s < NSTRIP // 2
        src_r = rlo_hbm if lo_half else rhi_hbm
        src_e = elo_hbm if lo_half else ehi_hbm
        src_g = glo_hbm if lo_half else ghi_hbm
        tok0 = (s % (NSTRIP // 2)) * STRIP
        pltpu.sync_copy(src_r.at[pl.ds(tok0, STRIP)], strip_r)
        pltpu.sync_copy(src_e.at[pl.ds(tok0, STRIP)], strip_e)
        if not is_bi:
            pltpu.sync_copy(src_g.at[pl.ds(tok0, STRIP)], strip_g)

        def body(k, carry):
            off = pl.multiple_of(k * LANES, 8)
            pos = plsc.load_gather(start_v, [strip_e[pl.ds(off, LANES)]]) \
                + strip_r[pl.ds(off, LANES)]
            if is_bi:
                plsc.store_scatter(dest_bi_v, [pos],
                                   iota16 + (tok0 + k * LANES))
            else:
                plsc.store_scatter(dest_gg_v, [pos],
                                   strip_g[pl.ds(off, LANES)])
            return carry

        lax.fori_loop(0, STRIP // LANES, body, 0)

    read_sems = (semr0, semr1)
    scat_sems = ((sema0, semb0), (sema1, semb1))

    def start_read(j, b):
        return pltpu.async_copy(
            net_hbm.at[pl.ds(base + j * CH, CH)], rows_v.at[b], read_sems[b])

    pending_read = {0: start_read(0, 0)}
    pending_scat = {}
    for j in range(NCHUNK):
        b = j % 2
        if j + 1 < NCHUNK:
            if j >= 1:
                for h in pending_scat.pop(j - 1):
                    h.wait()
            pending_read[j + 1] = start_read(j + 1, 1 - b)
        pending_read.pop(j).wait()
        s0, s1 = scat_sems[b]
        pending_scat[j] = (
            pltpu.async_copy(rows_v.at[b], ei_hbm.at[posrow_v.at[2 * j]], s0),
            pltpu.async_copy(rows_v.at[b],
                             ei_hbm.at[posrow_v.at[2 * j + 1]], s1),
        )
        if 2 <= j < 2 + NSTRIP:
            strip_idx = j - 2

            @pl.when(wid == 0)
            def _bi_strip():
                do_strip(strip_idx, True)

            @pl.when(wid == 1)
            def _gg_strip():
                do_strip(strip_idx, False)

    for j in sorted(pending_scat):
        for h in pending_scat[j]:
            h.wait()

    @pl.when(wid == 0)
    def _bi_out():
        pltpu.sync_copy(dest_bi_v, bi_hbm)

    @pl.when(wid == 1)
    def _gg_out():
        pltpu.sync_copy(dest_gg_v, gg_hbm)


def _sc_dispatch(net, start, rlo, rhi, elo, ehi, glo, ghi):
    mesh = plsc.VectorSubcoreMesh(core_axis_name="c", subcore_axis_name="s")
    fn = functools.partial(
        pl.kernel,
        out_type=(
            jax.ShapeDtypeStruct((N_TOK * TOP_K, D_MODEL), jnp.float32),
            jax.ShapeDtypeStruct((N_TOK * TOP_K,), jnp.int32),
            jax.ShapeDtypeStruct((N_TOK * TOP_K,), jnp.float32),
        ),
        mesh=mesh,
        compiler_params=pltpu.CompilerParams(needs_layout_passes=False),
        scratch_types=[
            pltpu.VMEM((NUM_EXPERTS,), jnp.int32),
            pltpu.VMEM((TOK_PER_W,), jnp.int32),
            pltpu.VMEM((TOK_PER_W,), jnp.int32),
            pltpu.VMEM((TOK_PER_W,), jnp.int32),
            pltpu.VMEM((TOK_PER_W,), jnp.int32),
            pltpu.VMEM((2 * NCHUNK, CH), jnp.int32),
            pltpu.VMEM((2, CH, D_MODEL), jnp.float32),
            pltpu.VMEM((STRIP,), jnp.int32),
            pltpu.VMEM((STRIP,), jnp.int32),
            pltpu.VMEM((STRIP,), jnp.float32),
            pltpu.VMEM((N_TOK * TOP_K,), jnp.int32),
            pltpu.VMEM((N_TOK * TOP_K,), jnp.float32),
            pltpu.SemaphoreType.DMA,
            pltpu.SemaphoreType.DMA,
            pltpu.SemaphoreType.DMA,
            pltpu.SemaphoreType.DMA,
            pltpu.SemaphoreType.DMA,
            pltpu.SemaphoreType.DMA,
        ],
    )(_sc_dispatch_body)
    return fn(net, start, rlo, rhi, elo, ehi, glo, ghi)


def kernel(net, logits, clean_logits, noise_std):
    (gates, load, part, hist, start,
     rlo, rhi, elo, ehi, glo, ghi) = _tc_route(logits, clean_logits,
                                               noise_std)
    del hist
    ei, bi, gg = _sc_dispatch(net, start, rlo, rhi, elo, ehi, glo, ghi)
    return gates, ei, bi, gg[:, None], load, part
